# baseline (speedup 1.0000x reference)
import functools

import jax
import jax.numpy as jnp
from jax import lax
from jax.experimental import pallas as pl
from jax.experimental.pallas import tpu as pltpu
from jax.experimental.pallas import tpu_sc as plsc

NUM_SAMPLES = 32
T = 128
CLIPS = 3
H = 224
W = 224
D = H * W
NC = 2
NS = 16


def _sc_subsample(x2):
    mesh = plsc.VectorSubcoreMesh(
        core_axis_name="c", subcore_axis_name="s", num_cores=NC, num_subcores=NS
    )

    @functools.partial(
        pl.kernel,
        out_type=jax.ShapeDtypeStruct((CLIPS * NUM_SAMPLES, D), jnp.float32),
        mesh=mesh,
        scratch_types=[
            pltpu.VMEM((D,), jnp.float32),
            pltpu.VMEM((D,), jnp.float32),
            pltpu.SemaphoreType.DMA,
            pltpu.SemaphoreType.DMA,
            pltpu.SemaphoreType.DMA,
            pltpu.SemaphoreType.DMA,
        ],
    )
    def body(x_hbm, out_hbm, buf0, buf1, in0, in1, out0, out1):
        cid = lax.axis_index("c")
        sid = lax.axis_index("s")
        wid = sid * NC + cid
        tsrc = (wid * (T - 1)) // (NUM_SAMPLES - 1)

        bufs = (buf0, buf1)
        in_sems = (in0, in1)
        out_sems = (out0, out1)

        in_dma0 = pltpu.async_copy(x_hbm.at[0 * T + tsrc], buf0, in0)
        in_dma1 = pltpu.async_copy(x_hbm.at[1 * T + tsrc], buf1, in1)
        in_dmas = [in_dma0, in_dma1]
        out_dmas = [None, None]

        for clip in range(CLIPS):
            slot = clip % 2
            in_dmas[slot].wait()
            out_dmas[slot] = pltpu.async_copy(
                bufs[slot], out_hbm.at[clip * NUM_SAMPLES + wid], out_sems[slot]
            )
            nxt = clip + 2
            if nxt < CLIPS:
                out_dmas[slot].wait()
                out_dmas[slot] = None
                in_dmas[slot] = pltpu.async_copy(
                    x_hbm.at[nxt * T + tsrc], bufs[slot], in_sems[slot]
                )

        for slot in range(2):
            if out_dmas[slot] is not None:
                out_dmas[slot].wait()

    return body(x2)


def kernel(x):
    x2 = x.reshape(CLIPS * T, D)
    out = _sc_subsample(x2)
    return out.reshape(CLIPS, NUM_SAMPLES, H, W)
